# baseline (speedup 1.0000x reference)
import jax
import jax.numpy as jnp
from jax import lax
from jax.experimental import pallas as pl
from jax.experimental.pallas import tpu as pltpu
from jax.experimental.pallas import tpu_sc as plsc

_BINS = 512
_NFIELD = 5
_FIELD = 20


def _split(a):
    hi = a.astype(jnp.bfloat16)
    lo = (a - hi.astype(jnp.float32)).astype(jnp.bfloat16)
    return hi, lo


def _dotb(a, b):
    return jnp.dot(a, b, preferred_element_type=jnp.float32)


def _dense_body(sub_ref, vf_ref, mrow_ref, mcol_ref, eoh_ref,
                w1ah_ref, w1al_ref, w1bh_ref, w1bl_ref, b1_ref,
                w2ah_ref, w2al_ref, b2a_ref, w2bh_ref, w2bl_ref, b2b_ref,
                wsh_ref, wsl_ref, bs_ref, probs_out):
    for j in range(sub_ref.shape[0]):
        _dense_one(j, sub_ref, vf_ref, mrow_ref, mcol_ref, eoh_ref,
                   w1ah_ref, w1al_ref, w1bh_ref, w1bl_ref, b1_ref,
                   w2ah_ref, w2al_ref, b2a_ref, w2bh_ref, w2bl_ref, b2b_ref,
                   wsh_ref, wsl_ref, bs_ref, probs_out)


def _dense_one(j, sub_ref, vf_ref, mrow_ref, mcol_ref, eoh_ref,
               w1ah_ref, w1al_ref, w1bh_ref, w1bl_ref, b1_ref,
               w2ah_ref, w2al_ref, b2a_ref, w2bh_ref, w2bl_ref, b2b_ref,
               wsh_ref, wsl_ref, bs_ref, probs_out):
    f32 = jnp.float32
    bf16 = jnp.bfloat16
    sub_i = sub_ref[j]
    sub = sub_i.astype(f32)
    vf = vf_ref[j]
    mrow = mrow_ref[j]
    mcol = mcol_ref[j]
    eoh = eoh_ref[j]

    vfm = vf * mcol
    asm = sub * mrow
    asm_b = asm.astype(bf16)
    vfh, vfl = _split(vfm)
    sws = _dotb(asm_b, vfh) + _dotb(asm_b, vfl)
    size = jnp.sum(asm, axis=1, keepdims=True) + 0.0001
    mu = jnp.mean(sws, axis=1, keepdims=True)
    var = jnp.mean((sws - mu) ** 2, axis=1, keepdims=True)
    z = (sws - mu) * lax.rsqrt(var + 1e-5 * (size * size))

    sub_b = sub.astype(bf16)
    pf = _dotb(sub_b, eoh.astype(bf16))
    nf, fw = _NFIELD, _FIELD
    ncols = nf * fw
    col_field = lax.broadcasted_iota(jnp.int32, (nf, ncols), 1) // fw
    row_id = lax.broadcasted_iota(jnp.int32, (nf, ncols), 0)
    expand = (col_field == row_id).astype(bf16)
    pfe = _dotb(pf.astype(bf16), expand)
    th = (lax.broadcasted_iota(jnp.int32, (1, ncols), 1) % fw).astype(f32)
    pf_oh_b = (th <= pfe).astype(bf16)

    zh, zl = _split(z)
    x = (_dotb(zh, w1ah_ref[...]) + _dotb(zh, w1al_ref[...])
         + _dotb(zl, w1ah_ref[...])
         + _dotb(pf_oh_b, w1bh_ref[...]) + _dotb(pf_oh_b, w1bl_ref[...])
         + b1_ref[...])
    x = jnp.maximum(x, 0.0)
    xb = x.astype(bf16)
    x = jnp.maximum(_dotb(xb, w2ah_ref[...]) + _dotb(xb, w2al_ref[...])
                    + b2a_ref[...], 0.0)
    xb = x.astype(bf16)
    x = jnp.maximum(_dotb(xb, w2bh_ref[...]) + _dotb(xb, w2bl_ref[...])
                    + b2b_ref[...], 0.0)
    mu2 = jnp.mean(x, axis=1, keepdims=True)
    var2 = jnp.mean((x - mu2) ** 2, axis=1, keepdims=True)
    z2 = (x - mu2) * lax.rsqrt(var2 + 1e-5)
    z2b = z2.astype(bf16)

    scores = _dotb(z2b, wsh_ref[...]) + _dotb(z2b, wsl_ref[...]) + bs_ref[...]
    m = jnp.max(scores, axis=0, keepdims=True)
    e = jnp.exp(scores - m)
    p = e / jnp.sum(e, axis=0, keepdims=True)
    probs_out[j] = p


def kernel(vert_feat_in, vert_mask_in, vert_element_oh, adj_oh, atom_subsets,
           atom_subsets_peaks, ln1_g, ln1_b, W1, b1, W2a, b2a, W2b, b2b,
           ln2_g, ln2_b, Ws, bs):
    B, A, GF = vert_feat_in.shape
    S = atom_subsets.shape[1]
    P = atom_subsets_peaks.shape[2]
    D = W1.shape[1]
    NE = vert_element_oh.shape[2]
    FE = W1.shape[0] - GF

    mrow = vert_mask_in.reshape(B, 1, A)
    mcol = vert_mask_in.reshape(B, A, 1)

    W1a = ln1_g[:, None] * W1[:GF]
    b1_eff = b1 + ln1_b @ W1[:GF]
    W1b = W1[GF:]
    Wse = ln2_g[:, None] * Ws
    bs_eff = bs + ln2_b @ Ws

    def sp(w):
        hi = w.astype(jnp.bfloat16)
        lo = (w - hi.astype(jnp.float32)).astype(jnp.bfloat16)
        return hi, lo

    w1ah, w1al = sp(W1a)
    w1bh, w1bl = sp(W1b)
    w2ah, w2al = sp(W2a)
    w2bh, w2bl = sp(W2b)
    wsh, wsl = sp(Wse)

    BB = 8

    def b3(shape):
        return pl.BlockSpec((BB,) + shape, lambda b: (b,) + (0,) * len(shape))

    def wspec(shape):
        return pl.BlockSpec(shape, lambda b: (0,) * len(shape))

    probs3 = pl.pallas_call(
        _dense_body,
        grid=(B // BB,),
        in_specs=[
            b3((S, A)),
            b3((A, GF)),
            b3((1, A)),
            b3((A, 1)),
            b3((A, NE)),
            wspec((GF, D)), wspec((GF, D)),
            wspec((FE, D)), wspec((FE, D)),
            wspec((1, D)),
            wspec((D, D)), wspec((D, D)), wspec((1, D)),
            wspec((D, D)), wspec((D, D)), wspec((1, D)),
            wspec((D, 1)), wspec((D, 1)), wspec((1, 1)),
        ],
        out_specs=pl.BlockSpec((BB, S, 1), lambda b: (b, 0, 0)),
        out_shape=jax.ShapeDtypeStruct((B, S, 1), jnp.float32),
    )(atom_subsets, vert_feat_in, mrow, mcol, vert_element_oh,
      w1ah, w1al, w1bh, w1bl, b1_eff.reshape(1, D),
      w2ah, w2al, b2a.reshape(1, D), w2bh, w2bl, b2b.reshape(1, D),
      wsh, wsl, bs_eff.reshape(1, 1))

    probs = probs3.reshape(B, S)

    nitems = S * P
    sp2 = nitems * 2
    peaks_flat = atom_subsets_peaks.reshape(B, sp2)

    def _hist_body(peaks_hbm, probs_hbm, out_hbm, peaks_v, probs_v, hist_v):
        f32 = jnp.float32
        wid = lax.axis_index("s") * 2 + lax.axis_index("c")
        pltpu.sync_copy(peaks_hbm.at[wid], peaks_v)
        pltpu.sync_copy(probs_hbm.at[wid], probs_v)
        zeros16 = jnp.zeros((16,), f32)
        iota16 = lax.iota(jnp.int32, 16)

        def zbody(i, c):
            hist_v[pl.ds(i * 16, 16)] = zeros16
            return c

        lax.fori_loop(0, _BINS // 16, zbody, 0)

        def body(i, c):
            lane = i * 16 + iota16
            mass = plsc.load_gather(peaks_v, [lane * 2])
            inten = plsc.load_gather(peaks_v, [lane * 2 + 1])
            pr = plsc.load_gather(probs_v, [lane // P])
            bn = jnp.clip((mass + 0.5).astype(jnp.int32), 0, _BINS - 1)
            plsc.addupdate_scatter(hist_v, [bn], inten * pr)
            return c

        lax.fori_loop(0, nitems // 16, body, 0)
        pltpu.sync_copy(hist_v, out_hbm.at[wid])

    spect = pl.kernel(
        _hist_body,
        mesh=plsc.VectorSubcoreMesh(core_axis_name="c", subcore_axis_name="s"),
        compiler_params=pltpu.CompilerParams(needs_layout_passes=False),
        out_type=jax.ShapeDtypeStruct((B, _BINS), jnp.float32),
        scratch_types=[
            pltpu.VMEM((sp2,), jnp.float32),
            pltpu.VMEM((S,), jnp.float32),
            pltpu.VMEM((_BINS,), jnp.float32),
        ],
    )(peaks_flat, probs)

    return (spect, probs)


# speedup vs baseline: 1.1807x; 1.1807x over previous
import jax
import jax.numpy as jnp
from jax import lax
from jax.experimental import pallas as pl
from jax.experimental.pallas import tpu as pltpu
from jax.experimental.pallas import tpu_sc as plsc

_BINS = 512
_NFIELD = 5
_FIELD = 20


def _split(a):
    hi = a.astype(jnp.bfloat16)
    lo = (a - hi.astype(jnp.float32)).astype(jnp.bfloat16)
    return hi, lo


def _dotb(a, b):
    return jnp.dot(a, b, preferred_element_type=jnp.float32)


def _dense_body(sub_ref, vf_ref, mrow_ref, mcol_ref, eoh_ref,
                w1ah_ref, w1al_ref, w1bh_ref, w1bl_ref, b1_ref,
                w2ah_ref, w2al_ref, b2a_ref, w2bh_ref, w2bl_ref, b2b_ref,
                wsh_ref, wsl_ref, bs_ref, probs_out):
    for j in range(sub_ref.shape[0]):
        _dense_one(j, sub_ref, vf_ref, mrow_ref, mcol_ref, eoh_ref,
                   w1ah_ref, w1al_ref, w1bh_ref, w1bl_ref, b1_ref,
                   w2ah_ref, w2al_ref, b2a_ref, w2bh_ref, w2bl_ref, b2b_ref,
                   wsh_ref, wsl_ref, bs_ref, probs_out)


def _dense_one(j, sub_ref, vf_ref, mrow_ref, mcol_ref, eoh_ref,
               w1ah_ref, w1al_ref, w1bh_ref, w1bl_ref, b1_ref,
               w2ah_ref, w2al_ref, b2a_ref, w2bh_ref, w2bl_ref, b2b_ref,
               wsh_ref, wsl_ref, bs_ref, probs_out):
    f32 = jnp.float32
    bf16 = jnp.bfloat16
    sub_i = sub_ref[j]
    sub = sub_i.astype(f32)
    vf = vf_ref[j]
    mrow = mrow_ref[j]
    mcol = mcol_ref[j]
    eoh = eoh_ref[j]

    vfm = vf * mcol
    asm = sub * mrow
    asm_b = asm.astype(bf16)
    vfh, vfl = _split(vfm)
    sws = _dotb(asm_b, vfh) + _dotb(asm_b, vfl)
    size = jnp.sum(asm, axis=1, keepdims=True) + 0.0001
    mu = jnp.mean(sws, axis=1, keepdims=True)
    var = jnp.mean((sws - mu) ** 2, axis=1, keepdims=True)
    z = (sws - mu) * lax.rsqrt(var + 1e-5 * (size * size))

    sub_b = sub.astype(bf16)
    pf = _dotb(sub_b, eoh.astype(bf16))
    nf, fw = _NFIELD, _FIELD
    ncols = nf * fw
    col_field = lax.broadcasted_iota(jnp.int32, (nf, ncols), 1) // fw
    row_id = lax.broadcasted_iota(jnp.int32, (nf, ncols), 0)
    expand = (col_field == row_id).astype(bf16)
    pfe = _dotb(pf.astype(bf16), expand)
    th = (lax.broadcasted_iota(jnp.int32, (1, ncols), 1) % fw).astype(f32)
    pf_oh_b = (th <= pfe).astype(bf16)

    zh, zl = _split(z)
    x = (_dotb(zh, w1ah_ref[...]) + _dotb(zh, w1al_ref[...])
         + _dotb(zl, w1ah_ref[...])
         + _dotb(pf_oh_b, w1bh_ref[...]) + _dotb(pf_oh_b, w1bl_ref[...])
         + b1_ref[...])
    x = jnp.maximum(x, 0.0)
    xb = x.astype(bf16)
    x = jnp.maximum(_dotb(xb, w2ah_ref[...]) + _dotb(xb, w2al_ref[...])
                    + b2a_ref[...], 0.0)
    xb = x.astype(bf16)
    x = jnp.maximum(_dotb(xb, w2bh_ref[...]) + _dotb(xb, w2bl_ref[...])
                    + b2b_ref[...], 0.0)
    mu2 = jnp.mean(x, axis=1, keepdims=True)
    var2 = jnp.mean((x - mu2) ** 2, axis=1, keepdims=True)
    z2 = (x - mu2) * lax.rsqrt(var2 + 1e-5)
    z2b = z2.astype(bf16)

    scores = _dotb(z2b, wsh_ref[...]) + _dotb(z2b, wsl_ref[...]) + bs_ref[...]
    m = jnp.max(scores, axis=0, keepdims=True)
    e = jnp.exp(scores - m)
    p = e / jnp.sum(e, axis=0, keepdims=True)
    probs_out[j] = p


def kernel(vert_feat_in, vert_mask_in, vert_element_oh, adj_oh, atom_subsets,
           atom_subsets_peaks, ln1_g, ln1_b, W1, b1, W2a, b2a, W2b, b2b,
           ln2_g, ln2_b, Ws, bs):
    B, A, GF = vert_feat_in.shape
    S = atom_subsets.shape[1]
    P = atom_subsets_peaks.shape[2]
    D = W1.shape[1]
    NE = vert_element_oh.shape[2]
    FE = W1.shape[0] - GF

    mrow = vert_mask_in.reshape(B, 1, A)
    mcol = vert_mask_in.reshape(B, A, 1)

    W1a = ln1_g[:, None] * W1[:GF]
    b1_eff = b1 + ln1_b @ W1[:GF]
    W1b = W1[GF:]
    Wse = ln2_g[:, None] * Ws
    bs_eff = bs + ln2_b @ Ws

    def sp(w):
        hi = w.astype(jnp.bfloat16)
        lo = (w - hi.astype(jnp.float32)).astype(jnp.bfloat16)
        return hi, lo

    w1ah, w1al = sp(W1a)
    w1bh, w1bl = sp(W1b)
    w2ah, w2al = sp(W2a)
    w2bh, w2bl = sp(W2b)
    wsh, wsl = sp(Wse)

    BB = 4

    def b3(shape):
        return pl.BlockSpec((BB,) + shape, lambda b: (b,) + (0,) * len(shape))

    def wspec(shape):
        return pl.BlockSpec(shape, lambda b: (0,) * len(shape))

    probs3 = pl.pallas_call(
        _dense_body,
        grid=(B // BB,),
        in_specs=[
            b3((S, A)),
            b3((A, GF)),
            b3((1, A)),
            b3((A, 1)),
            b3((A, NE)),
            wspec((GF, D)), wspec((GF, D)),
            wspec((FE, D)), wspec((FE, D)),
            wspec((1, D)),
            wspec((D, D)), wspec((D, D)), wspec((1, D)),
            wspec((D, D)), wspec((D, D)), wspec((1, D)),
            wspec((D, 1)), wspec((D, 1)), wspec((1, 1)),
        ],
        out_specs=pl.BlockSpec((BB, S, 1), lambda b: (b, 0, 0)),
        out_shape=jax.ShapeDtypeStruct((B, S, 1), jnp.float32),
    )(atom_subsets, vert_feat_in, mrow, mcol, vert_element_oh,
      w1ah, w1al, w1bh, w1bl, b1_eff.reshape(1, D),
      w2ah, w2al, b2a.reshape(1, D), w2bh, w2bl, b2b.reshape(1, D),
      wsh, wsl, bs_eff.reshape(1, 1))

    probs = probs3.reshape(B, S)

    nitems = S * P
    sp2 = nitems * 2
    peaks_flat = atom_subsets_peaks.reshape(B, sp2)

    def _hist_body(peaks_hbm, probs_hbm, out_hbm, peaks_v, probs_v, hist_v):
        f32 = jnp.float32
        wid = lax.axis_index("s") * 2 + lax.axis_index("c")
        pltpu.sync_copy(peaks_hbm.at[wid], peaks_v)
        pltpu.sync_copy(probs_hbm.at[wid], probs_v)
        zeros16 = jnp.zeros((16,), f32)
        iota16 = lax.iota(jnp.int32, 16)

        def zbody(i, c):
            hist_v[pl.ds(i * 16, 16)] = zeros16
            return c

        lax.fori_loop(0, _BINS // 16, zbody, 0)

        def body(i, c):
            lane = i * 16 + iota16
            mass = plsc.load_gather(peaks_v, [lane * 2])
            inten = plsc.load_gather(peaks_v, [lane * 2 + 1])
            pr = plsc.load_gather(probs_v, [lane // P])
            bn = jnp.clip((mass + 0.5).astype(jnp.int32), 0, _BINS - 1)
            plsc.addupdate_scatter(hist_v, [bn], inten * pr)
            return c

        lax.fori_loop(0, nitems // 16, body, 0)
        pltpu.sync_copy(hist_v, out_hbm.at[wid])

    spect = pl.kernel(
        _hist_body,
        mesh=plsc.VectorSubcoreMesh(core_axis_name="c", subcore_axis_name="s"),
        compiler_params=pltpu.CompilerParams(needs_layout_passes=False),
        out_type=jax.ShapeDtypeStruct((B, _BINS), jnp.float32),
        scratch_types=[
            pltpu.VMEM((sp2,), jnp.float32),
            pltpu.VMEM((S,), jnp.float32),
            pltpu.VMEM((_BINS,), jnp.float32),
        ],
    )(peaks_flat, probs)

    return (spect, probs)
